# Initial kernel scaffold; baseline (speedup 1.0000x reference)
import jax
import jax.numpy as jnp
from jax.experimental import pallas as pl


def kernel(inputs, tables):
    raise NotImplementedError("write your pallas kernel here")


# speedup vs baseline: 7.5079x; 7.5079x over previous
import jax
import jax.numpy as jnp
from jax import lax
from jax.experimental import pallas as pl
from jax.experimental.pallas import tpu as pltpu
from jax.experimental.pallas import tpu_sc as plsc

N_FEATURES = 26
INPUT_DIM = 33
OUT_DIM = 32
BATCH = 16384

NC, NS, L = 2, 16, 16
NW = NC * NS
TOTAL = BATCH * N_FEATURES
PER_W = TOTAL // NW
CHUNK = 1664
N_CHUNKS = PER_W // CHUNK
G = 128
NG = CHUNK // G


def _embed_body(idx_hbm, off_hbm, tab_hbm, out_hbm,
                idx_v, off_v, flat_v, rows_v, sem):
    wid = lax.axis_index("s") * NC + lax.axis_index("c")
    wbase = wid * PER_W
    pltpu.sync_copy(off_hbm, off_v)

    for c in range(N_CHUNKS):
        base = wbase + c * CHUNK
        pltpu.sync_copy(idx_hbm.at[pl.ds(base, CHUNK)], idx_v)

        def step(i, carry):
            r = i // (G // L)
            col = (i % (G // L)) * L
            v = idx_v[pl.ds(i * L, L)] + off_v[pl.ds(i * L, L)]
            flat_v[r, pl.ds(col, L)] = v
            return carry

        lax.fori_loop(0, CHUNK // L, step, 0)

        copies = [
            pltpu.make_async_copy(
                tab_hbm.at[flat_v.at[g]],
                rows_v.at[pl.ds(g * G, G)],
                sem,
            )
            for g in range(NG)
        ]
        for cp in copies:
            cp.start()
        for cp in copies:
            cp.wait()

        pltpu.sync_copy(rows_v, out_hbm.at[pl.ds(base, CHUNK)])


def kernel(inputs, tables):
    idx_flat = inputs.reshape(TOTAL)
    tab_flat = tables.reshape(N_FEATURES * INPUT_DIM, OUT_DIM)
    off = jnp.tile(
        jnp.arange(N_FEATURES, dtype=jnp.int32) * INPUT_DIM,
        CHUNK // N_FEATURES,
    )

    run = pl.kernel(
        _embed_body,
        out_type=jax.ShapeDtypeStruct((TOTAL, OUT_DIM), jnp.float32),
        mesh=plsc.VectorSubcoreMesh(core_axis_name="c", subcore_axis_name="s"),
        scratch_types=[
            pltpu.VMEM((CHUNK,), jnp.int32),
            pltpu.VMEM((CHUNK,), jnp.int32),
            pltpu.VMEM((NG, G), jnp.int32),
            pltpu.VMEM((CHUNK, OUT_DIM), jnp.float32),
            pltpu.SemaphoreType.DMA,
        ],
        compiler_params=pltpu.CompilerParams(use_tc_tiling_on_sc=False),
    )
    out = run(idx_flat, off, tab_flat)
    return out.reshape(BATCH, N_FEATURES * OUT_DIM)


# speedup vs baseline: 7.5903x; 1.0110x over previous
import jax
import jax.numpy as jnp
from jax import lax
from jax.experimental import pallas as pl
from jax.experimental.pallas import tpu as pltpu
from jax.experimental.pallas import tpu_sc as plsc

N_FEATURES = 26
INPUT_DIM = 33
OUT_DIM = 32
BATCH = 16384

NC, NS, L = 2, 16, 16
NW = NC * NS
TOTAL = BATCH * N_FEATURES
PER_W = TOTAL // NW
G = 128
N_GROUPS = PER_W // G
CHUNK = 1664
NG = CHUNK // G
N_CHUNKS = PER_W // CHUNK
OFF_LEN = 208


def _embed_body(idx_hbm, off_hbm, tab_hbm, out_hbm,
                idx_v, off_v, rows0, rows1, sg0, sg1, sw0, sw1):
    wid = lax.axis_index("s") * NC + lax.axis_index("c")
    wbase = wid * PER_W
    pltpu.sync_copy(off_hbm, off_v)
    pltpu.sync_copy(idx_hbm.at[pl.ds(wbase // G, N_GROUPS)], idx_v)

    def step(i, carry):
        r = i // (G // L)
        col = (i % (G // L)) * L
        off = off_v[pl.ds((i % (OFF_LEN // L)) * L, L)]
        idx_v[r, pl.ds(col, L)] = idx_v[r, pl.ds(col, L)] + off
        return carry

    lax.fori_loop(0, PER_W // L, step, 0)

    bufs = (rows0, rows1)
    gsems = (sg0, sg1)
    wsems = (sw0, sw1)
    pend_g = [None, None]
    pend_w = [None, None]

    for c in range(N_CHUNKS + 1):
        if c < N_CHUNKS:
            b = c % 2
            if pend_w[b] is not None:
                pend_w[b].wait()
            gs = []
            for g in range(NG):
                cp = pltpu.make_async_copy(
                    tab_hbm.at[idx_v.at[c * NG + g]],
                    bufs[b].at[pl.ds(g * G, G)],
                    gsems[b],
                )
                cp.start()
                gs.append(cp)
            pend_g[b] = gs
        if c >= 1:
            b2 = (c - 1) % 2
            for cp in pend_g[b2]:
                cp.wait()
            wr = pltpu.make_async_copy(
                bufs[b2],
                out_hbm.at[pl.ds(wbase + (c - 1) * CHUNK, CHUNK)],
                wsems[b2],
            )
            wr.start()
            pend_w[b2] = wr

    pend_w[(N_CHUNKS - 1) % 2].wait()


def kernel(inputs, tables):
    idx_flat = inputs.reshape(TOTAL // G, G)
    tab_flat = tables.reshape(N_FEATURES * INPUT_DIM, OUT_DIM)
    off = jnp.tile(
        jnp.arange(N_FEATURES, dtype=jnp.int32) * INPUT_DIM,
        OFF_LEN // N_FEATURES,
    )

    run = pl.kernel(
        _embed_body,
        out_type=jax.ShapeDtypeStruct((TOTAL, OUT_DIM), jnp.float32),
        mesh=plsc.VectorSubcoreMesh(core_axis_name="c", subcore_axis_name="s"),
        scratch_types=[
            pltpu.VMEM((N_GROUPS, G), jnp.int32),
            pltpu.VMEM((OFF_LEN,), jnp.int32),
            pltpu.VMEM((CHUNK, OUT_DIM), jnp.float32),
            pltpu.VMEM((CHUNK, OUT_DIM), jnp.float32),
            pltpu.SemaphoreType.DMA,
            pltpu.SemaphoreType.DMA,
            pltpu.SemaphoreType.DMA,
            pltpu.SemaphoreType.DMA,
        ],
        compiler_params=pltpu.CompilerParams(use_tc_tiling_on_sc=False),
    )
    out = run(idx_flat, off, tab_flat)
    return out.reshape(BATCH, N_FEATURES * OUT_DIM)
